# Initial kernel scaffold; baseline (speedup 1.0000x reference)
import jax
import jax.numpy as jnp
from jax import lax
from jax.experimental import pallas as pl
from jax.experimental.pallas import tpu as pltpu
from jax.experimental.pallas import tpu_sc as plsc

N_USER_ = 50000
N_ITEM_ = 50000
N_ = N_USER_ + N_ITEM_
D_ = 32
K_ = 10
E_ = 1600000

NC_ = 2
NS_ = 16
HALF_ = N_ // NC_
ACC_R_ = 51200
TRASH_ = HALF_
EBLK_ = 128
PAD_E_ = 16 * EBLK_ * 782
EB_ = PAD_E_ // NS_
NB_E_ = EB_ // EBLK_
NBLK_N_ = HALF_ // 16


def _body(emb_h, src_h, dst_h, val_h, nbr_h, wts_h, out_h,
          acc, zbuf, sidx, didx, lidx, vbuf, rows,
          nidx, wbuf, nrows, ebuf, abuf, obuf):
    c = lax.axis_index("c")
    s = lax.axis_index("s")
    cbase = c * HALF_

    zero16 = jnp.zeros((16,), jnp.float32)

    @pl.loop(0, EBLK_)
    def _(r):
        zbuf[r, pl.ds(0, 16)] = zero16
        zbuf[r, pl.ds(16, 16)] = zero16

    @pl.loop(0, ACC_R_ // NS_ // EBLK_)
    def _(j):
        row0 = pl.multiple_of(s * (ACC_R_ // NS_) + j * EBLK_, EBLK_)
        pltpu.sync_copy(zbuf, acc.at[pl.ds(row0, EBLK_)])

    plsc.subcore_barrier()

    ebase0 = s * EB_

    @pl.loop(0, NB_E_)
    def _(b):
        ebase = pl.multiple_of(ebase0 + b * EBLK_, EBLK_)
        pltpu.sync_copy(src_h.at[pl.ds(ebase, EBLK_)], sidx)
        pltpu.sync_copy(dst_h.at[pl.ds(ebase, EBLK_)], didx)
        pltpu.sync_copy(val_h.at[pl.ds(ebase, EBLK_)], vbuf)
        pltpu.sync_copy(emb_h.at[sidx], rows)

        @pl.loop(0, 8)
        def _(g):
            off = pl.multiple_of(g * 16, 16)
            d = didx[pl.ds(off, 16)]
            loc = d - cbase
            ok = (loc >= 0) & (loc < HALF_)
            lidx[pl.ds(off, 16)] = jnp.where(ok, loc, TRASH_)
            for e in range(16):
                r = off + e
                bc = plsc.load_gather(vbuf, [jnp.full((16,), r, jnp.int32)])
                rows[r, pl.ds(0, 16)] = rows[r, pl.ds(0, 16)] * bc
                rows[r, pl.ds(16, 16)] = rows[r, pl.ds(16, 16)] * bc

        pltpu.sync_copy(rows, acc.at[lidx], add=True)

    plsc.subcore_barrier()

    @pl.loop(0, (NBLK_N_ + NS_ - 1) // NS_)
    def _(j):
        blk = s + j * NS_

        @pl.when(blk < NBLK_N_)
        def _():
            lbase = pl.multiple_of(blk * 16, 16)
            gbase = pl.multiple_of(cbase + lbase, 16)
            ibase = pl.multiple_of(gbase * K_, 32)
            pltpu.sync_copy(nbr_h.at[pl.ds(ibase, 16 * K_)], nidx)
            pltpu.sync_copy(wts_h.at[pl.ds(ibase, 16 * K_)], wbuf)
            pltpu.sync_copy(emb_h.at[nidx.at[pl.ds(0, 128)]],
                            nrows.at[pl.ds(0, 128)])
            pltpu.sync_copy(emb_h.at[nidx.at[pl.ds(128, 32)]],
                            nrows.at[pl.ds(128, 32)])
            pltpu.sync_copy(emb_h.at[pl.ds(gbase, 16)], ebuf)
            pltpu.sync_copy(acc.at[pl.ds(lbase, 16)], abuf)

            @pl.loop(0, 16)
            def _(n):
                a0 = ebuf[n, pl.ds(0, 16)] + abuf[n, pl.ds(0, 16)]
                a1 = ebuf[n, pl.ds(16, 16)] + abuf[n, pl.ds(16, 16)]
                for k in range(K_):
                    r = n * K_ + k
                    bc = plsc.load_gather(
                        wbuf, [jnp.full((16,), r, jnp.int32)])
                    a0 = a0 + nrows[r, pl.ds(0, 16)] * bc
                    a1 = a1 + nrows[r, pl.ds(16, 16)] * bc
                obuf[n, pl.ds(0, 16)] = a0
                obuf[n, pl.ds(16, 16)] = a1

            pltpu.sync_copy(obuf, out_h.at[pl.ds(gbase, 16)])


def kernel(user_emb, item_emb, user_sim_neighbor, user_sim_weight,
           item_sim_neighbor, item_sim_weight, graph_edge_index, graph_values):
    emb = jnp.concatenate([user_emb, item_emb], axis=0)
    pad = PAD_E_ - E_
    src = jnp.concatenate([graph_edge_index[1],
                           jnp.zeros((pad,), jnp.int32)])
    dst = jnp.concatenate([graph_edge_index[0],
                           jnp.zeros((pad,), jnp.int32)])
    val = jnp.concatenate([graph_values, jnp.zeros((pad,), jnp.float32)])
    nbr = jnp.concatenate([user_sim_neighbor.reshape(-1),
                           item_sim_neighbor.reshape(-1) + N_USER_])
    wts = jnp.concatenate([user_sim_weight.reshape(-1),
                           item_sim_weight.reshape(-1)])

    mesh = plsc.VectorSubcoreMesh(core_axis_name="c", subcore_axis_name="s",
                                  num_cores=NC_, num_subcores=NS_)
    run = pl.kernel(
        _body,
        out_type=jax.ShapeDtypeStruct((N_, D_), jnp.float32),
        mesh=mesh,
        scratch_types=[
            pltpu.VMEM_SHARED((ACC_R_, D_), jnp.float32),
            pltpu.VMEM((EBLK_, D_), jnp.float32),
            pltpu.VMEM((EBLK_,), jnp.int32),
            pltpu.VMEM((EBLK_,), jnp.int32),
            pltpu.VMEM((EBLK_,), jnp.int32),
            pltpu.VMEM((EBLK_,), jnp.float32),
            pltpu.VMEM((EBLK_, D_), jnp.float32),
            pltpu.VMEM((16 * K_,), jnp.int32),
            pltpu.VMEM((16 * K_,), jnp.float32),
            pltpu.VMEM((16 * K_, D_), jnp.float32),
            pltpu.VMEM((16, D_), jnp.float32),
            pltpu.VMEM((16, D_), jnp.float32),
            pltpu.VMEM((16, D_), jnp.float32),
        ],
    )
    out = run(emb, src, dst, val, nbr, wts)
    return (out[:N_USER_], out[N_USER_:])


# speedup vs baseline: 3.9294x; 3.9294x over previous
import dataclasses

import jax
import jax.numpy as jnp
from jax import lax
from jax.experimental import pallas as pl
from jax.experimental.pallas import tpu as pltpu
from jax.experimental.pallas import tpu_sc as plsc

N_USER_ = 50000
N_ITEM_ = 50000
N_ = N_USER_ + N_ITEM_
D_ = 32
K_ = 10
E_ = 1600000

NC_ = 2
NS_ = 16
HALF_ = N_ // NC_
ACC_R_ = 51200
TRASH_ = HALF_
EBLK_ = 128
PAD_E_ = 16 * EBLK_ * 782
EB_ = PAD_E_ // NS_
NB_E_ = EB_ // EBLK_
NBLK_N_ = HALF_ // 16


def _body(emb_h, src_h, dst_h, val_h, nbr_h, wts_h, out_h,
          acc, zbuf, sidx, didx, lidx, vbuf, rows,
          nidx, wbuf, nrows, ebuf, abuf, obuf):
    c = lax.axis_index("c")
    s = lax.axis_index("s")
    cbase = c * HALF_

    zero16 = jnp.zeros((16,), jnp.float32)

    @pl.loop(0, EBLK_)
    def _(r):
        zbuf[r, pl.ds(0, 16)] = zero16
        zbuf[r, pl.ds(16, 16)] = zero16

    @pl.loop(0, ACC_R_ // NS_ // EBLK_)
    def _(j):
        row0 = pl.multiple_of(s * (ACC_R_ // NS_) + j * EBLK_, EBLK_)
        pltpu.sync_copy(zbuf, acc.at[pl.ds(row0, EBLK_)])

    plsc.subcore_barrier()

    ebase0 = s * EB_

    @pl.loop(0, NB_E_)
    def _(b):
        ebase = pl.multiple_of(ebase0 + b * EBLK_, EBLK_)
        pltpu.sync_copy(src_h.at[pl.ds(ebase, EBLK_)], sidx)
        pltpu.sync_copy(dst_h.at[pl.ds(ebase, EBLK_)], didx)
        pltpu.sync_copy(val_h.at[pl.ds(ebase, EBLK_)], vbuf)
        pltpu.sync_copy(emb_h.at[sidx], rows)

        @pl.loop(0, 8)
        def _(g):
            off = pl.multiple_of(g * 16, 16)
            d = didx[pl.ds(off, 16)]
            loc = d - cbase
            ok = (loc >= 0) & (loc < HALF_)
            lidx[pl.ds(off, 16)] = jnp.where(ok, loc, TRASH_)
            for e in range(16):
                r = off + e
                bc = plsc.load_gather(vbuf, [jnp.full((16,), r, jnp.int32)])
                rows[r, pl.ds(0, 16)] = rows[r, pl.ds(0, 16)] * bc
                rows[r, pl.ds(16, 16)] = rows[r, pl.ds(16, 16)] * bc

        pltpu.sync_copy(rows, acc.at[lidx], add=True)

    plsc.subcore_barrier()

    @pl.loop(0, (NBLK_N_ + NS_ - 1) // NS_)
    def _(j):
        blk = s + j * NS_

        @pl.when(blk < NBLK_N_)
        def _():
            lbase = pl.multiple_of(blk * 16, 16)
            gbase = pl.multiple_of(cbase + lbase, 16)
            ibase = pl.multiple_of(gbase * K_, 32)
            pltpu.sync_copy(nbr_h.at[pl.ds(ibase, 16 * K_)], nidx)
            pltpu.sync_copy(wts_h.at[pl.ds(ibase, 16 * K_)], wbuf)
            pltpu.sync_copy(emb_h.at[nidx.at[pl.ds(0, 128)]],
                            nrows.at[pl.ds(0, 128)])
            pltpu.sync_copy(emb_h.at[nidx.at[pl.ds(128, 32)]],
                            nrows.at[pl.ds(128, 32)])
            pltpu.sync_copy(emb_h.at[pl.ds(gbase, 16)], ebuf)
            pltpu.sync_copy(acc.at[pl.ds(lbase, 16)], abuf)

            @pl.loop(0, 16)
            def _(n):
                a0 = ebuf[n, pl.ds(0, 16)] + abuf[n, pl.ds(0, 16)]
                a1 = ebuf[n, pl.ds(16, 16)] + abuf[n, pl.ds(16, 16)]
                for k in range(K_):
                    r = n * K_ + k
                    bc = plsc.load_gather(
                        wbuf, [jnp.full((16,), r, jnp.int32)])
                    a0 = a0 + nrows[r, pl.ds(0, 16)] * bc
                    a1 = a1 + nrows[r, pl.ds(16, 16)] * bc
                obuf[n, pl.ds(0, 16)] = a0
                obuf[n, pl.ds(16, 16)] = a1

            pltpu.sync_copy(obuf, out_h.at[pl.ds(gbase, 16)])


def kernel(user_emb, item_emb, user_sim_neighbor, user_sim_weight,
           item_sim_neighbor, item_sim_weight, graph_edge_index, graph_values):
    emb = jnp.concatenate([user_emb, item_emb], axis=0)
    pad = PAD_E_ - E_
    src = jnp.concatenate([graph_edge_index[1],
                           jnp.zeros((pad,), jnp.int32)])
    dst = jnp.concatenate([graph_edge_index[0],
                           jnp.zeros((pad,), jnp.int32)])
    val = jnp.concatenate([graph_values, jnp.zeros((pad,), jnp.float32)])
    nbr = jnp.concatenate([user_sim_neighbor.reshape(-1),
                           item_sim_neighbor.reshape(-1) + N_USER_])
    wts = jnp.concatenate([user_sim_weight.reshape(-1),
                           item_sim_weight.reshape(-1)])

    mesh = plsc.VectorSubcoreMesh(core_axis_name="c", subcore_axis_name="s",
                                  num_cores=NC_, num_subcores=NS_)
    cp = pltpu.CompilerParams()
    if "needs_layout_passes" in pltpu.CompilerParams.__dataclass_fields__:
        cp = dataclasses.replace(cp, needs_layout_passes=False)
    if "use_tc_tiling_on_sc" in pltpu.CompilerParams.__dataclass_fields__:
        cp = dataclasses.replace(cp, use_tc_tiling_on_sc=False)
    run = pl.kernel(
        _body,
        out_type=jax.ShapeDtypeStruct((N_, D_), jnp.float32),
        mesh=mesh,
        scratch_types=[
            pltpu.VMEM_SHARED((ACC_R_, D_), jnp.float32),
            pltpu.VMEM((EBLK_, D_), jnp.float32),
            pltpu.VMEM((EBLK_,), jnp.int32),
            pltpu.VMEM((EBLK_,), jnp.int32),
            pltpu.VMEM((EBLK_,), jnp.int32),
            pltpu.VMEM((EBLK_,), jnp.float32),
            pltpu.VMEM((EBLK_, D_), jnp.float32),
            pltpu.VMEM((16 * K_,), jnp.int32),
            pltpu.VMEM((16 * K_,), jnp.float32),
            pltpu.VMEM((16 * K_, D_), jnp.float32),
            pltpu.VMEM((16, D_), jnp.float32),
            pltpu.VMEM((16, D_), jnp.float32),
            pltpu.VMEM((16, D_), jnp.float32),
        ],
        compiler_params=cp,
    )
    out = run(emb, src, dst, val, nbr, wts)
    return (out[:N_USER_], out[N_USER_:])
